# baseline (speedup 1.0000x reference)
import functools

import jax
import jax.numpy as jnp
from jax import lax
from jax.experimental import pallas as pl
from jax.experimental.pallas import tpu as pltpu
from jax.experimental.pallas import tpu_sc as plsc

N = 10000
D = 128
E = 320000
OUT = 10
L = 4

NC, NS = 2, 16
NW = NC * NS
RPT = 80
EPAD = NW * RPT * 128
NB = 2
NH = RPT // 2
NP = 10112
RT = NP // NS
DUMMY = N

_BN = 2000
_GRID = N // _BN


@functools.cache
def _make_sc_kernels():
    mesh = plsc.VectorSubcoreMesh(core_axis_name="c", subcore_axis_name="s",
                                  num_cores=NC, num_subcores=NS)

    @functools.partial(
        pl.kernel,
        out_type=jax.ShapeDtypeStruct((NC * NP, D), jnp.float32),
        mesh=mesh,
        scratch_types=[
            pltpu.VMEM((128,), jnp.int32),
            pltpu.VMEM((128,), jnp.int32),
            pltpu.VMEM((128, D), jnp.float32),
            pltpu.VMEM_SHARED((NP, D), jnp.float32),
            pltpu.SemaphoreType.DMA,
        ],
    )
    def _sc_prop(xt, srcg, dsts, zeros, out, srcv, dstv, rows, acc, sem):
        c = lax.axis_index("c")
        s = lax.axis_index("s")
        w = c * NS + s
        r0 = s * RT
        pltpu.sync_copy(zeros.at[pl.ds(r0, RT)], acc.at[pl.ds(r0, RT)])
        plsc.subcore_barrier()

        def step(k, carry):
            row = w * RPT + k
            pltpu.sync_copy(srcg.at[row], srcv)
            pltpu.sync_copy(dsts.at[row], dstv)
            pltpu.async_copy(xt.at[srcv], rows, sem).wait()
            pltpu.sync_copy(rows, acc.at[dstv], add=True)
            return carry

        lax.fori_loop(0, RPT, step, 0)
        plsc.subcore_barrier()
        pltpu.sync_copy(acc.at[pl.ds(r0, RT)], out.at[pl.ds(c * NP + r0, RT)])

    @functools.partial(
        pl.kernel,
        out_type=jax.ShapeDtypeStruct((NC * NP, 16), jnp.float32),
        mesh=mesh,
        scratch_types=[
            pltpu.VMEM((128,), jnp.int32),
            pltpu.VMEM((128, 16), jnp.float32),
            pltpu.VMEM_SHARED((NP, 16), jnp.float32),
        ],
    )
    def _sc_deg(srcd, zeros16, ones16, out, idxv, ones, acc):
        c = lax.axis_index("c")
        s = lax.axis_index("s")
        w = c * NS + s
        r0 = s * RT
        pltpu.sync_copy(ones16, ones)
        pltpu.sync_copy(zeros16.at[pl.ds(r0, RT)], acc.at[pl.ds(r0, RT)])
        plsc.subcore_barrier()

        def step(k, carry):
            pltpu.sync_copy(srcd.at[w * RPT + k], idxv)
            pltpu.sync_copy(ones, acc.at[idxv], add=True)
            return carry

        lax.fori_loop(0, RPT, step, 0)
        plsc.subcore_barrier()
        pltpu.sync_copy(acc.at[pl.ds(r0, RT)], out.at[pl.ds(c * NP + r0, RT)])

    return _sc_prop, _sc_deg


def _row_spec(width=D):
    return pl.BlockSpec((_BN, width), lambda i: (i, 0))


def _pair_spec(width=D):
    return pl.BlockSpec((2, _BN, width), lambda i: (0, i, 0))


def _full_spec(shape):
    return pl.BlockSpec(shape, lambda i: tuple(0 for _ in shape))


def _prep_body(dp_ref, feat_ref, xt_ref, dis_ref, dis2_ref):
    deg = dp_ref[0, :, 0:1] + dp_ref[1, :, 0:1]
    dis = jnp.where(deg > 0, lax.rsqrt(jnp.maximum(deg, 1e-12)), 0.0)
    xt_ref[...] = dis * feat_ref[...]
    dis_ref[...] = dis
    dis2_ref[...] = dis * dis


def _make_prep(interpret=False):
    return pl.pallas_call(
        _prep_body,
        grid=(_GRID,),
        in_specs=[_pair_spec(16), _row_spec()],
        out_specs=[_row_spec(), _row_spec(1), _row_spec(1)],
        out_shape=[
            jax.ShapeDtypeStruct((N, D), jnp.float32),
            jax.ShapeDtypeStruct((N, 1), jnp.float32),
            jax.ShapeDtypeStruct((N, 1), jnp.float32),
        ],
        interpret=interpret,
    )


def _scale1_body(p_ref, dis2_ref, a1_ref):
    a1_ref[...] = -dis2_ref[...] * (p_ref[0] + p_ref[1])


def _make_scale1(interpret=False):
    return pl.pallas_call(
        _scale1_body,
        grid=(_GRID,),
        in_specs=[_pair_spec(), _row_spec(1)],
        out_specs=_row_spec(),
        out_shape=jax.ShapeDtypeStruct((N, D), jnp.float32),
        interpret=interpret,
    )


def _scale2_body(p_ref, dis2_ref, a0_ref, a2_ref):
    a2_ref[...] = -2.0 * dis2_ref[...] * (p_ref[0] + p_ref[1]) - a0_ref[...]


def _make_scale2(interpret=False):
    return pl.pallas_call(
        _scale2_body,
        grid=(_GRID,),
        in_specs=[_pair_spec(), _row_spec(1), _row_spec()],
        out_specs=_row_spec(),
        out_shape=jax.ShapeDtypeStruct((N, D), jnp.float32),
        interpret=interpret,
    )


def _mm_body(x_ref, p1_ref, p2_ref, p3_ref, dis_ref, wp_ref, b_ref,
             y_ref, stats_ref, acc_ref):
    i = pl.program_id(0)
    dis = dis_ref[...]
    x = x_ref[...]
    z1 = dis * (p1_ref[0] + p1_ref[1])
    z2 = dis * (p2_ref[0] + p2_ref[1])
    z3 = dis * (p3_ref[0] + p3_ref[1])
    y = jnp.dot(x, wp_ref[0], preferred_element_type=jnp.float32)
    y += jnp.dot(z1, wp_ref[1], preferred_element_type=jnp.float32)
    y += jnp.dot(z2, wp_ref[2], preferred_element_type=jnp.float32)
    y += jnp.dot(z3, wp_ref[3], preferred_element_type=jnp.float32)
    y += b_ref[...]
    y_ref[...] = y

    @pl.when(i == 0)
    def _():
        acc_ref[...] = jnp.zeros_like(acc_ref)

    acc_ref[0:1, :] += jnp.sum(y, axis=0, keepdims=True)
    acc_ref[1:2, :] += jnp.sum(y * y, axis=0, keepdims=True)

    @pl.when(i == _GRID - 1)
    def _():
        stats_ref[...] = acc_ref[...]


def _make_mm(interpret=False):
    return pl.pallas_call(
        _mm_body,
        grid=(_GRID,),
        in_specs=[_row_spec(), _pair_spec(), _pair_spec(), _pair_spec(),
                  _row_spec(1), _full_spec((4, D, D)), _full_spec((1, D))],
        out_specs=[_row_spec(), _full_spec((2, D))],
        out_shape=[
            jax.ShapeDtypeStruct((N, D), jnp.float32),
            jax.ShapeDtypeStruct((2, D), jnp.float32),
        ],
        scratch_shapes=[pltpu.VMEM((2, D), jnp.float32)],
        interpret=interpret,
    )


def _gn(y, stats, w, b, ms):
    mean = stats[0:1, :] * (1.0 / N)
    ey2 = stats[1:2, :] * (1.0 / N)
    var = ey2 - (2.0 * ms - ms * ms) * mean * mean
    return w * (y - ms * mean) * lax.rsqrt(var + 1e-5) + b


def _norm_body(y_ref, stats_ref, w_ref, b_ref, ms_ref, dis_ref,
               xn_ref, a0_ref):
    v = _gn(y_ref[...], stats_ref[...], w_ref[...], b_ref[...], ms_ref[...])
    v = jnp.where(v > 0, v, 0.1 * v)
    xn_ref[...] = v
    a0_ref[...] = dis_ref[...] * v


def _make_norm(interpret=False):
    return pl.pallas_call(
        _norm_body,
        grid=(_GRID,),
        in_specs=[_row_spec(), _full_spec((2, D)), _full_spec((1, D)),
                  _full_spec((1, D)), _full_spec((1, D)), _row_spec(1)],
        out_specs=[_row_spec(), _row_spec()],
        out_shape=[
            jax.ShapeDtypeStruct((N, D), jnp.float32),
            jax.ShapeDtypeStruct((N, D), jnp.float32),
        ],
        interpret=interpret,
    )


def _norm3_body(y_ref, stats_ref, w_ref, b_ref, ms_ref, feat_ref,
                psum_ref, acc_ref):
    i = pl.program_id(0)
    v = _gn(y_ref[...], stats_ref[...], w_ref[...], b_ref[...], ms_ref[...])
    f = jnp.maximum(feat_ref[...] + v, 0.0)

    @pl.when(i == 0)
    def _():
        acc_ref[...] = jnp.zeros_like(acc_ref)

    acc_ref[...] += jnp.sum(f, axis=0, keepdims=True)

    @pl.when(i == _GRID - 1)
    def _():
        psum_ref[...] = acc_ref[...]


def _make_norm3(interpret=False):
    return pl.pallas_call(
        _norm3_body,
        grid=(_GRID,),
        in_specs=[_row_spec(), _full_spec((2, D)), _full_spec((1, D)),
                  _full_spec((1, D)), _full_spec((1, D)), _row_spec()],
        out_specs=_full_spec((1, D)),
        out_shape=jax.ShapeDtypeStruct((1, D), jnp.float32),
        scratch_shapes=[pltpu.VMEM((1, D), jnp.float32)],
        interpret=interpret,
    )


def _final_body(psum_ref, lw_ref, lb_ref, out_ref):
    pooled = psum_ref[...] * (1.0 / N)
    o = lax.dot_general(pooled, lw_ref[...], (((1,), (1,)), ((), ())),
                        preferred_element_type=jnp.float32)
    o = o + lb_ref[...]
    out_ref[...] = jnp.maximum(o, 0.0) + jnp.log(1.0 + jnp.exp(-jnp.abs(o)))


def _make_final(interpret=False):
    return pl.pallas_call(
        _final_body,
        in_specs=[pl.BlockSpec((1, D), lambda: (0, 0)),
                  pl.BlockSpec((OUT, D), lambda: (0, 0)),
                  pl.BlockSpec((1, OUT), lambda: (0, 0))],
        out_specs=pl.BlockSpec((1, OUT), lambda: (0, 0)),
        out_shape=jax.ShapeDtypeStruct((1, OUT), jnp.float32),
        interpret=interpret,
    )


def kernel(edge_index, feat, convW, convB, gnW, gnB, gnMS, linW, linB):
    src = edge_index[0].astype(jnp.int32)
    dst = edge_index[1].astype(jnp.int32)
    pad = EPAD - E
    nrows = EPAD // 128
    srcg = jnp.concatenate([src, jnp.zeros((pad,), jnp.int32)]).reshape(nrows, 128)
    dsts = jnp.concatenate([dst, jnp.full((pad,), DUMMY, jnp.int32)]).reshape(nrows, 128)
    srcd = jnp.concatenate([src, jnp.full((pad,), DUMMY, jnp.int32)]).reshape(nrows, 128)
    zeros = jnp.zeros((NP, D), jnp.float32)
    zeros16 = jnp.zeros((NP, 16), jnp.float32)
    ones16 = jnp.ones((128, 16), jnp.float32)

    convW = convW.astype(jnp.float32)
    Wp = jnp.stack([
        convW[:, 0] - convW[:, 2],
        convW[:, 3] - convW[:, 1],
        -2.0 * convW[:, 2],
        -2.0 * convW[:, 3],
    ], axis=1)

    _sc_prop, _sc_deg = _make_sc_kernels()
    prep = _make_prep()
    scale1 = _make_scale1()
    scale2 = _make_scale2()
    mm = _make_mm()
    norm = _make_norm()
    norm3 = _make_norm3()
    final = _make_final()

    dp = _sc_deg(srcd, zeros16, ones16).reshape(2, NP, 16)
    xt0, dis, dis2 = prep(dp, feat)

    x = feat
    a0 = xt0
    for i in range(L):
        p1 = _sc_prop(a0, srcg, dsts, zeros).reshape(2, NP, D)
        a1 = scale1(p1, dis2)
        p2 = _sc_prop(a1, srcg, dsts, zeros).reshape(2, NP, D)
        a2 = scale2(p2, dis2, a0)
        p3 = _sc_prop(a2, srcg, dsts, zeros).reshape(2, NP, D)
        y, stats = mm(x, p1, p2, p3, dis, Wp[i],
                      convB[i].reshape(1, D))
        if i < L - 1:
            x, a0 = norm(y, stats, gnW[i].reshape(1, D), gnB[i].reshape(1, D),
                         gnMS[i].reshape(1, D), dis)
        else:
            psum = norm3(y, stats, gnW[i].reshape(1, D), gnB[i].reshape(1, D),
                         gnMS[i].reshape(1, D), feat)
    out = final(psum, linW, linB.reshape(1, OUT))
    return out[0]


# speedup vs baseline: 2.0725x; 2.0725x over previous
import functools

import jax
import jax.numpy as jnp
from jax import lax
from jax.experimental import pallas as pl
from jax.experimental.pallas import tpu as pltpu
from jax.experimental.pallas import tpu_sc as plsc

N = 10000
D = 128
E = 320000
OUT = 10
L = 4

NC, NS = 2, 16
NW = NC * NS
RPT = 80
EPAD = NW * RPT * 128
NB = 2
NH = RPT // 2
NP = 10112
RT = NP // NS
DUMMY = N

_BN = 2000
_GRID = N // _BN


@functools.cache
def _make_sc_kernels():
    mesh = plsc.VectorSubcoreMesh(core_axis_name="c", subcore_axis_name="s",
                                  num_cores=NC, num_subcores=NS)

    @functools.partial(
        pl.kernel,
        out_type=jax.ShapeDtypeStruct((NC * NP, D), jnp.float32),
        mesh=mesh,
        scratch_types=[
            pltpu.VMEM((128,), jnp.int32),
            pltpu.VMEM((128,), jnp.int32),
            pltpu.VMEM((128, D), jnp.float32),
            pltpu.VMEM_SHARED((NP, D), jnp.float32),
            pltpu.SemaphoreType.DMA,
        ],
    )
    def _sc_prop(xt, srcg, dsts, zeros, out, srcv, dstv, rows, acc, sem):
        c = lax.axis_index("c")
        s = lax.axis_index("s")
        w = c * NS + s
        r0 = s * RT
        pltpu.sync_copy(zeros.at[pl.ds(r0, RT)], acc.at[pl.ds(r0, RT)])
        plsc.subcore_barrier()

        def step(k, carry):
            row = w * RPT + k
            pltpu.sync_copy(srcg.at[row], srcv)
            pltpu.sync_copy(dsts.at[row], dstv)
            pltpu.async_copy(xt.at[srcv], rows, sem).wait()
            pltpu.sync_copy(rows, acc.at[dstv], add=True)
            return carry

        lax.fori_loop(0, RPT, step, 0)
        plsc.subcore_barrier()
        pltpu.sync_copy(acc.at[pl.ds(r0, RT)], out.at[pl.ds(c * NP + r0, RT)])

    @functools.partial(
        pl.kernel,
        out_type=jax.ShapeDtypeStruct((NC * NP, 16), jnp.float32),
        mesh=mesh,
        scratch_types=[
            pltpu.VMEM((128,), jnp.int32),
            pltpu.VMEM((128, 16), jnp.float32),
            pltpu.VMEM_SHARED((NP, 16), jnp.float32),
        ],
    )
    def _sc_deg(srcd, zeros16, ones16, out, idxv, ones, acc):
        c = lax.axis_index("c")
        s = lax.axis_index("s")
        w = c * NS + s
        r0 = s * RT
        pltpu.sync_copy(ones16, ones)
        pltpu.sync_copy(zeros16.at[pl.ds(r0, RT)], acc.at[pl.ds(r0, RT)])
        plsc.subcore_barrier()

        def step(k, carry):
            pltpu.sync_copy(srcd.at[w * RPT + k], idxv)
            pltpu.sync_copy(ones, acc.at[idxv], add=True)
            return carry

        lax.fori_loop(0, RPT, step, 0)
        plsc.subcore_barrier()
        pltpu.sync_copy(acc.at[pl.ds(r0, RT)], out.at[pl.ds(c * NP + r0, RT)])

    return _sc_prop, _sc_deg


def _row_spec(width=D):
    return pl.BlockSpec((_BN, width), lambda i: (i, 0))


def _pair_spec(width=D):
    return pl.BlockSpec((2, _BN, width), lambda i: (0, i, 0))


def _full_spec(shape):
    return pl.BlockSpec(shape, lambda i: tuple(0 for _ in shape))


def _prep_body(dp_ref, feat_ref, xt_ref, dis_ref, dis2_ref):
    deg = dp_ref[0, :, 0:1] + dp_ref[1, :, 0:1]
    dis = jnp.where(deg > 0, lax.rsqrt(jnp.maximum(deg, 1e-12)), 0.0)
    xt_ref[...] = dis * feat_ref[...]
    dis_ref[...] = dis
    dis2_ref[...] = dis * dis


def _make_prep(interpret=False):
    return pl.pallas_call(
        _prep_body,
        grid=(_GRID,),
        in_specs=[_pair_spec(16), _row_spec()],
        out_specs=[_row_spec(), _row_spec(1), _row_spec(1)],
        out_shape=[
            jax.ShapeDtypeStruct((N, D), jnp.float32),
            jax.ShapeDtypeStruct((N, 1), jnp.float32),
            jax.ShapeDtypeStruct((N, 1), jnp.float32),
        ],
        interpret=interpret,
    )


def _scale1_body(p_ref, dis2_ref, a1_ref):
    a1_ref[...] = -dis2_ref[...] * (p_ref[0] + p_ref[1])


def _make_scale1(interpret=False):
    return pl.pallas_call(
        _scale1_body,
        grid=(_GRID,),
        in_specs=[_pair_spec(), _row_spec(1)],
        out_specs=_row_spec(),
        out_shape=jax.ShapeDtypeStruct((N, D), jnp.float32),
        interpret=interpret,
    )


def _scale2_body(p_ref, dis2_ref, a0_ref, a2_ref):
    a2_ref[...] = -2.0 * dis2_ref[...] * (p_ref[0] + p_ref[1]) - a0_ref[...]


def _make_scale2(interpret=False):
    return pl.pallas_call(
        _scale2_body,
        grid=(_GRID,),
        in_specs=[_pair_spec(), _row_spec(1), _row_spec()],
        out_specs=_row_spec(),
        out_shape=jax.ShapeDtypeStruct((N, D), jnp.float32),
        interpret=interpret,
    )


def _mm_body(x_ref, p1_ref, p2_ref, p3_ref, dis_ref, wp_ref, b_ref,
             y_ref, stats_ref, acc_ref):
    i = pl.program_id(0)
    dis = dis_ref[...]
    x = x_ref[...]
    z1 = dis * (p1_ref[0] + p1_ref[1])
    z2 = dis * (p2_ref[0] + p2_ref[1])
    z3 = dis * (p3_ref[0] + p3_ref[1])
    y = jnp.dot(x, wp_ref[0], preferred_element_type=jnp.float32)
    y += jnp.dot(z1, wp_ref[1], preferred_element_type=jnp.float32)
    y += jnp.dot(z2, wp_ref[2], preferred_element_type=jnp.float32)
    y += jnp.dot(z3, wp_ref[3], preferred_element_type=jnp.float32)
    y += b_ref[...]
    y_ref[...] = y

    @pl.when(i == 0)
    def _():
        acc_ref[...] = jnp.zeros_like(acc_ref)

    acc_ref[0:1, :] += jnp.sum(y, axis=0, keepdims=True)
    acc_ref[1:2, :] += jnp.sum(y * y, axis=0, keepdims=True)

    @pl.when(i == _GRID - 1)
    def _():
        stats_ref[...] = acc_ref[...]


def _make_mm(interpret=False):
    return pl.pallas_call(
        _mm_body,
        grid=(_GRID,),
        in_specs=[_row_spec(), _pair_spec(), _pair_spec(), _pair_spec(),
                  _row_spec(1), _full_spec((4, D, D)), _full_spec((1, D))],
        out_specs=[_row_spec(), _full_spec((2, D))],
        out_shape=[
            jax.ShapeDtypeStruct((N, D), jnp.float32),
            jax.ShapeDtypeStruct((2, D), jnp.float32),
        ],
        scratch_shapes=[pltpu.VMEM((2, D), jnp.float32)],
        interpret=interpret,
    )


def _gn(y, stats, w, b, ms):
    mean = stats[0:1, :] * (1.0 / N)
    ey2 = stats[1:2, :] * (1.0 / N)
    var = ey2 - (2.0 * ms - ms * ms) * mean * mean
    return w * (y - ms * mean) * lax.rsqrt(var + 1e-5) + b


def _norm_body(y_ref, stats_ref, w_ref, b_ref, ms_ref, dis_ref,
               xn_ref, a0_ref):
    v = _gn(y_ref[...], stats_ref[...], w_ref[...], b_ref[...], ms_ref[...])
    v = jnp.where(v > 0, v, 0.1 * v)
    xn_ref[...] = v
    a0_ref[...] = dis_ref[...] * v


def _make_norm(interpret=False):
    return pl.pallas_call(
        _norm_body,
        grid=(_GRID,),
        in_specs=[_row_spec(), _full_spec((2, D)), _full_spec((1, D)),
                  _full_spec((1, D)), _full_spec((1, D)), _row_spec(1)],
        out_specs=[_row_spec(), _row_spec()],
        out_shape=[
            jax.ShapeDtypeStruct((N, D), jnp.float32),
            jax.ShapeDtypeStruct((N, D), jnp.float32),
        ],
        interpret=interpret,
    )


def _norm3_body(y_ref, stats_ref, w_ref, b_ref, ms_ref, feat_ref,
                psum_ref, acc_ref):
    i = pl.program_id(0)
    v = _gn(y_ref[...], stats_ref[...], w_ref[...], b_ref[...], ms_ref[...])
    f = jnp.maximum(feat_ref[...] + v, 0.0)

    @pl.when(i == 0)
    def _():
        acc_ref[...] = jnp.zeros_like(acc_ref)

    acc_ref[...] += jnp.sum(f, axis=0, keepdims=True)

    @pl.when(i == _GRID - 1)
    def _():
        psum_ref[...] = acc_ref[...]


def _make_norm3(interpret=False):
    return pl.pallas_call(
        _norm3_body,
        grid=(_GRID,),
        in_specs=[_row_spec(), _full_spec((2, D)), _full_spec((1, D)),
                  _full_spec((1, D)), _full_spec((1, D)), _row_spec()],
        out_specs=_full_spec((1, D)),
        out_shape=jax.ShapeDtypeStruct((1, D), jnp.float32),
        scratch_shapes=[pltpu.VMEM((1, D), jnp.float32)],
        interpret=interpret,
    )


def _final_body(psum_ref, lw_ref, lb_ref, out_ref):
    pooled = psum_ref[...] * (1.0 / N)
    o = lax.dot_general(pooled, lw_ref[...], (((1,), (1,)), ((), ())),
                        preferred_element_type=jnp.float32)
    o = o + lb_ref[...]
    out_ref[...] = jnp.maximum(o, 0.0) + jnp.log(1.0 + jnp.exp(-jnp.abs(o)))


def _make_final(interpret=False):
    return pl.pallas_call(
        _final_body,
        in_specs=[pl.BlockSpec((1, D), lambda: (0, 0)),
                  pl.BlockSpec((OUT, D), lambda: (0, 0)),
                  pl.BlockSpec((1, OUT), lambda: (0, 0))],
        out_specs=pl.BlockSpec((1, OUT), lambda: (0, 0)),
        out_shape=jax.ShapeDtypeStruct((1, OUT), jnp.float32),
        interpret=interpret,
    )


def kernel(edge_index, feat, convW, convB, gnW, gnB, gnMS, linW, linB):
    src = edge_index[0].astype(jnp.int32)
    dst = edge_index[1].astype(jnp.int32)
    pad = EPAD - E
    nrows = EPAD // 128
    padi = jnp.arange(pad, dtype=jnp.int32)
    pad_src = padi % jnp.int32(N)
    pad_dst = jnp.int32(N) + padi % jnp.int32(NP - N)
    srcg = jnp.concatenate([src, pad_src]).reshape(nrows, 128)
    dsts = jnp.concatenate([dst, pad_dst]).reshape(nrows, 128)
    srcd = jnp.concatenate([src, pad_dst]).reshape(nrows, 128)
    zeros = jnp.zeros((NP, D), jnp.float32)
    zeros16 = jnp.zeros((NP, 16), jnp.float32)
    ones16 = jnp.ones((128, 16), jnp.float32)

    convW = convW.astype(jnp.float32)
    Wp = jnp.stack([
        convW[:, 0] - convW[:, 2],
        convW[:, 3] - convW[:, 1],
        -2.0 * convW[:, 2],
        -2.0 * convW[:, 3],
    ], axis=1)

    _sc_prop, _sc_deg = _make_sc_kernels()
    prep = _make_prep()
    scale1 = _make_scale1()
    scale2 = _make_scale2()
    mm = _make_mm()
    norm = _make_norm()
    norm3 = _make_norm3()
    final = _make_final()

    dp = _sc_deg(srcd, zeros16, ones16).reshape(2, NP, 16)
    xt0, dis, dis2 = prep(dp, feat)

    x = feat
    a0 = xt0
    for i in range(L):
        p1 = _sc_prop(a0, srcg, dsts, zeros).reshape(2, NP, D)
        a1 = scale1(p1, dis2)
        p2 = _sc_prop(a1, srcg, dsts, zeros).reshape(2, NP, D)
        a2 = scale2(p2, dis2, a0)
        p3 = _sc_prop(a2, srcg, dsts, zeros).reshape(2, NP, D)
        y, stats = mm(x, p1, p2, p3, dis, Wp[i],
                      convB[i].reshape(1, D))
        if i < L - 1:
            x, a0 = norm(y, stats, gnW[i].reshape(1, D), gnB[i].reshape(1, D),
                         gnMS[i].reshape(1, D), dis)
        else:
            psum = norm3(y, stats, gnW[i].reshape(1, D), gnB[i].reshape(1, D),
                         gnMS[i].reshape(1, D), feat)
    out = final(psum, linW, linB.reshape(1, OUT))
    return out[0]
